# baseline (speedup 1.0000x reference)
import dataclasses
import functools

import jax
import jax.numpy as jnp
from jax.experimental import pallas as pl
from jax.experimental.pallas import tpu as pltpu
from jax.experimental.pallas import tpu_sc as plsc

SD = 64
T = 4096
NB = 16
N0 = T * NB
HORIZON = 8


@functools.cache
def _sc_mesh():
    return plsc.VectorSubcoreMesh(core_axis_name="c", subcore_axis_name="s")


@functools.cache
def _sc_params():
    cp = pltpu.CompilerParams(use_tc_tiling_on_sc=False)
    if "needs_layout_passes" in pltpu.CompilerParams.__dataclass_fields__:
        cp = dataclasses.replace(cp, needs_layout_passes=False)
    return cp


def _row_dot(row, mat):
    return jax.lax.dot_general(row, mat, (((1,), (1,)), ((), ())))


def _thresh(cv, key_o, meta_o, offs_o):
    s = jax.lax.bitcast_convert_type(cv, jnp.int32)
    key = s ^ ((s >> 31) & jnp.int32(0x7FFFFFFF))
    key_o[...] = key
    msb = jnp.int32(-2147483648)

    def bit(i, pu):
        cand_u = pu | (jnp.int32(1) << (31 - i))
        cand_s = cand_u ^ msb
        cnt = jnp.sum((key >= cand_s).astype(jnp.int32))
        return jnp.where(cnt >= T, cand_u, pu)

    pu = jax.lax.fori_loop(0, 32, bit, jnp.int32(0))
    t = pu ^ msb
    gt = key > t
    eq = key == t
    rem = T - jnp.sum(gt.astype(jnp.int32))
    ii = (jax.lax.broadcasted_iota(jnp.int32, (NB, T), 0) * T
          + jax.lax.broadcasted_iota(jnp.int32, (NB, T), 1))

    def bit2(i, p0):
        cand = p0 | (jnp.int32(1) << (16 - i))
        c = jnp.sum((eq & (ii < cand)).astype(jnp.int32))
        return jnp.where(c < rem, cand, p0)

    p0 = jax.lax.fori_loop(0, 17, bit2, jnp.int32(0))
    mask = gt | (eq & (ii <= p0))
    mf = mask.astype(jnp.float32)
    rowcnt = jnp.dot(mf, jnp.ones((T, 1), jnp.float32))
    lower = (jax.lax.broadcasted_iota(jnp.int32, (NB, NB), 1)
             < jax.lax.broadcasted_iota(jnp.int32, (NB, NB), 0))
    offs_o[...] = jnp.dot(lower.astype(jnp.float32), rowcnt).astype(jnp.int32)
    meta_o[0] = t
    meta_o[1] = p0
    for q in range(2, 16):
        meta_o[q] = jnp.int32(0)


def _preamble_body(s_ref, adv_ref, wm1, wm2, wa1, wa2, wp1, wp2, whs, whc,
                   strat_o, sa_o, sp_o, pi_o, psw_o, h0_o):
    s = s_ref[...]
    adv = adv_ref[...]
    strat = jnp.tanh(adv @ wm1[...] + s @ wm2[...])
    ps = jnp.tanh(strat @ wa1[...] + s @ wa2[...])
    pi = jnp.tanh(strat @ wp1[...] + ps @ wp2[...])
    strat_o[...] = strat
    sa_o[...] = strat @ wa1[...]
    sp_o[...] = strat @ wp1[...]
    pi_o[...] = pi
    psw_o[...] = ps @ whs[...]
    h0_o[...] = s @ whc[...]


def _bloom_body(pi_ref, psw_ref, strat_c, whc, wha, wv, h0_ref, nz_ref,
                cns_o, cv_o, meta_o, offs_o, cv_sc):
    j = pl.program_id(0)
    ca = jnp.tanh(pi_ref[...] + nz_ref[...])
    cns = jnp.tanh(psw_ref[...] + ca @ wha[...])
    cns_o[...] = cns
    v = cns @ whc[...] - h0_ref[...]
    vp = (jnp.tanh(cns @ wv[...]) @ strat_c[...]) * ((HORIZON - 1.0) / HORIZON)
    cv_sc[pl.ds(j, 1), :] = (v + vp).reshape(1, T)

    @pl.when(j == NB - 1)
    def _():
        _thresh(cv_sc[...], cv_o, meta_o, offs_o)


def _round_body(last, sa, sp, wa2, wp2, whs, whc, wha, wv,
                strat_c, h0_ref, nz_ref, cst_ref, *refs):
    if last:
        cv_o, cv_sc, pi_ref, psw_ref = refs
    else:
        ns_o, cv_o, meta_o, offs_o, cv_sc, pi_ref, psw_ref = refs
    j = pl.program_id(0)
    scale = (HORIZON - 2.0 - (1.0 if last else 0.0)) / HORIZON

    @pl.when(j == 0)
    def _():
        ps_b = jnp.tanh(sa[...] + cst_ref[...] @ wa2[...])
        pi_ref[...] = jnp.tanh(sp[...] + ps_b @ wp2[...])
        psw_ref[...] = ps_b @ whs[...]

    @pl.when(j > 0)
    def _():
        a = jnp.tanh(pi_ref[...] + nz_ref[0])
        ns = jnp.tanh(psw_ref[...] + a @ wha[...])
        if not last:
            ns_o[...] = ns
        v = ns @ whc[...] - h0_ref[...]
        vp = (jnp.tanh(ns @ wv[...]) @ strat_c[...]) * scale
        cv_sc[pl.ds(jnp.maximum(j - 1, 0), 1), :] = (v + vp).reshape(1, T)

    @pl.when(j == NB)
    def _():
        if last:
            s = jax.lax.bitcast_convert_type(cv_sc[...], jnp.int32)
            cv_o[...] = s ^ ((s >> 31) & jnp.int32(0x7FFFFFFF))
        else:
            _thresh(cv_sc[...], cv_o, meta_o, offs_o)


def _sc_select(first):
    def body(cv_hbm, meta_hbm, offs_hbm, anc_hbm, cns_hbm, cst_o, anc_o,
             kv, mv, ov, av, sb, ab, sp, ap, accs, acca, rows, spad, apad,
             s1, s2, s3, s4):
        cid = jax.lax.axis_index("c")
        sid = jax.lax.axis_index("s")

        @pl.when(cid == 0)
        def _():
            base = sid * 4096
            c1 = pltpu.async_copy(cv_hbm.at[sid], kv, s1)
            c2 = pltpu.async_copy(meta_hbm, mv, s2)
            c3 = pltpu.async_copy(offs_hbm, ov.at[pl.ds(0, 16)], s3)
            if not first:
                c4 = pltpu.async_copy(anc_hbm, av, s4)
            lane = jax.lax.iota(jnp.int32, 16)
            zero = lane - lane

            @pl.loop(0, 260)
            def _(c):
                sb[pl.ds(c * 16, 16)] = zero
                ab[pl.ds(c * 16, 16)] = zero

            c1.wait()
            c2.wait()
            c3.wait()
            if not first:
                c4.wait()
            mvv = mv[...]
            t = mvv[0]
            p0 = mvv[1]
            my_off = ov[pl.ds(sid, 16)][0]

            def chunk(c, run):
                off = c * 16
                k16 = kv[pl.ds(off, 16)]
                gidx = lane + (base + off)
                m = (k16 > t) | ((k16 == t) & (gidx <= p0))
                mi = m.astype(jnp.int32)
                inc = plsc.cumsum(mi)
                dst = jnp.where(m, run + (inc - mi), T + lane)
                plsc.store_scatter(sb, [dst], gidx, mask=m)
                if first:
                    av_ = gidx
                else:
                    av_ = plsc.load_gather(av, [gidx & (T - 1)])
                plsc.store_scatter(ab, [dst], av_, mask=m)
                return run + jnp.sum(mi)

            jax.lax.fori_loop(0, 256, chunk, my_off)
            c5 = pltpu.async_copy(sb, spad.at[sid], s1)
            c6 = pltpu.async_copy(ab, apad.at[sid], s2)
            c5.wait()
            c6.wait()
            plsc.subcore_barrier()

            off2 = sid * 256
            c7 = pltpu.async_copy(spad.at[:, pl.ds(off2, 256)], sp, s1)
            c8 = pltpu.async_copy(apad.at[:, pl.ds(off2, 256)], ap, s2)
            c7.wait()
            c8.wait()

            @pl.loop(0, 16)
            def _(c):
                o16 = c * 16
                s = sp[0, pl.ds(o16, 16)]
                a = ap[0, pl.ds(o16, 16)]
                for r in range(1, 16):
                    s = s + sp[r, pl.ds(o16, 16)]
                    a = a + ap[r, pl.ds(o16, 16)]
                accs[pl.ds(o16, 16)] = s
                acca[pl.ds(o16, 16)] = a

            c9 = pltpu.async_copy(acca, anc_o.at[pl.ds(off2, 256)], s3)
            pltpu.sync_copy(cns_hbm.at[accs], rows)
            pltpu.sync_copy(rows, cst_o.at[pl.ds(off2, 256)])
            c9.wait()

    return body


def _final_body(cv_ref, anc_ref, pi_ref, nz_ref, out_o, row, sem):
    x = cv_ref[...]
    m = jnp.max(x)
    ii = (jax.lax.broadcasted_iota(jnp.int32, (NB, T), 0) * T
          + jax.lax.broadcasted_iota(jnp.int32, (NB, T), 1))
    flat = jnp.min(jnp.where(x == m, ii, jnp.int32(2 ** 30)))
    g = anc_ref[flat % T]
    cp = pltpu.make_async_copy(nz_ref.at[pl.ds(g, 1), :], row, sem)
    cp.start()
    cp.wait()
    out_o[...] = jnp.tanh(pi_ref[...] + row[...])


def _select(cv16, meta, offs, anc, cns, first):
    i32 = jnp.int32
    cst, anc_new = pl.kernel(
        _sc_select(first),
        out_type=[jax.ShapeDtypeStruct((T, SD), jnp.float32),
                  jax.ShapeDtypeStruct((T,), i32)],
        mesh=_sc_mesh(),
        compiler_params=_sc_params(),
        scratch_types=[pltpu.VMEM((4096,), i32),
                       pltpu.VMEM((16,), i32),
                       pltpu.VMEM((32,), i32),
                       pltpu.VMEM((T,), i32),
                       pltpu.VMEM((4160,), i32),
                       pltpu.VMEM((4160,), i32),
                       pltpu.VMEM((16, 256), i32),
                       pltpu.VMEM((16, 256), i32),
                       pltpu.VMEM((256,), i32),
                       pltpu.VMEM((256,), i32),
                       pltpu.VMEM((256, SD), jnp.float32),
                       pltpu.VMEM_SHARED((16, 4160), i32),
                       pltpu.VMEM_SHARED((16, 4160), i32),
                       pltpu.SemaphoreType.DMA,
                       pltpu.SemaphoreType.DMA,
                       pltpu.SemaphoreType.DMA,
                       pltpu.SemaphoreType.DMA],
    )(cv16, meta, offs.reshape(NB), anc, cns)
    return cst, anc_new


def kernel(s_t, adversary_strategy, W_m1, W_m2, W_a1, W_a2, W_p1, W_p2,
           W_h_a, W_h_s, W_v, w_health, noise):
    call = pl.pallas_call
    f32 = jnp.float32
    i32 = jnp.int32
    s2 = s_t.reshape(1, SD)
    adv2 = adversary_strategy.reshape(1, SD)
    whc = w_health.reshape(SD, 1)

    vec = jax.ShapeDtypeStruct((1, SD), f32)
    rep = pl.BlockSpec((1, SD), lambda j: (0, 0))
    rep_c = pl.BlockSpec((SD, 1), lambda j: (0, 0))
    rep_m = pl.BlockSpec((SD, SD), lambda j: (0, 0))
    full16 = pl.BlockSpec((NB, T), lambda j: (0, 0))
    smem = pl.BlockSpec(memory_space=pltpu.SMEM)

    strat, strat_a, strat_p, pi, psw, h0 = call(
        _preamble_body,
        out_shape=[vec, vec, vec, vec, vec, jax.ShapeDtypeStruct((1, 1), f32)],
    )(s2, adv2, W_m1, W_m2, W_a1, W_a2, W_p1, W_p2, W_h_s, whc)
    strat_c = strat.reshape(SD, 1)

    cns, cv16, meta, offs = call(
        _bloom_body,
        grid=(NB,),
        in_specs=[rep, rep, rep_c, rep_c, rep_m, rep_m,
                  pl.BlockSpec((1, 1), lambda j: (0, 0)),
                  pl.BlockSpec((T, SD), lambda j: (j, 0))],
        out_specs=[pl.BlockSpec((T, SD), lambda j: (j, 0)), full16, smem,
                   pl.BlockSpec((NB, 1), lambda j: (0, 0))],
        out_shape=[jax.ShapeDtypeStruct((N0, SD), f32),
                   jax.ShapeDtypeStruct((NB, T), i32),
                   jax.ShapeDtypeStruct((16,), i32),
                   jax.ShapeDtypeStruct((NB, 1), i32)],
        scratch_shapes=[pltpu.VMEM((NB, T), f32)],
    )(pi, psw, strat_c, whc, W_h_a, W_v, h0, noise)

    noise16 = noise[:NB].reshape(NB, 1, SD)
    anc = jnp.zeros((T,), i32)
    for rnd in (1, 2):
        cst, anc = _select(cv16, meta, offs, anc, cns, rnd == 1)
        last = rnd == 2
        nz_spec = pl.BlockSpec((1, 1, SD),
                               lambda j: (jnp.maximum(j - 1, 0), 0, 0))
        outs = call(
            functools.partial(_round_body, last),
            grid=(NB + 1,),
            in_specs=[rep, rep, rep_m, rep_m, rep_m, rep_c, rep_m, rep_m,
                      rep_c, pl.BlockSpec((1, 1), lambda j: (0, 0)), nz_spec,
                      pl.BlockSpec((T, SD), lambda j: (0, 0))],
            out_specs=([full16] if last else
                       [pl.BlockSpec((T, SD),
                                     lambda j: (jnp.maximum(j - 1, 0), 0)),
                        full16, smem, pl.BlockSpec((NB, 1), lambda j: (0, 0))]),
            out_shape=([jax.ShapeDtypeStruct((NB, T), i32)] if last else
                       [jax.ShapeDtypeStruct((N0, SD), f32),
                        jax.ShapeDtypeStruct((NB, T), i32),
                        jax.ShapeDtypeStruct((16,), i32),
                        jax.ShapeDtypeStruct((NB, 1), i32)]),
            scratch_shapes=[pltpu.VMEM((NB, T), f32),
                            pltpu.VMEM((T, SD), f32),
                            pltpu.VMEM((T, SD), f32)],
        )(strat_a, strat_p, W_a2, W_p2, W_h_s, whc, W_h_a, W_v, strat_c, h0,
          noise16, cst)
        if last:
            cv16 = outs if isinstance(outs, jax.Array) else outs[0]
        else:
            cns, cv16, meta, offs = outs

    out = call(
        _final_body,
        in_specs=[pl.BlockSpec((NB, T), lambda: (0, 0)), smem,
                  pl.BlockSpec((1, SD), lambda: (0, 0)),
                  pl.BlockSpec(memory_space=pl.ANY)],
        out_specs=pl.BlockSpec((1, SD), lambda: (0, 0)),
        out_shape=jax.ShapeDtypeStruct((1, SD), f32),
        scratch_shapes=[pltpu.VMEM((1, SD), f32), pltpu.SemaphoreType.DMA],
    )(cv16, anc, pi, noise)
    return out.reshape(SD)
